# initial kernel scaffold (unmeasured)
import jax
import jax.numpy as jnp
from jax import lax
from jax.experimental import pallas as pl
from jax.experimental.pallas import tpu as pltpu

N_DEV = 4
N_TOK = 4096
D_MODEL = 1024
H = 2048
E_LOCAL = 4
CHUNK = N_TOK // N_DEV
N_STEPS = 2 * (N_DEV - 1)


def kernel(x, router_W, route_idx, expert_W):
    del router_W
    xb = x.astype(jnp.bfloat16)
    wb = expert_W.astype(jnp.bfloat16)

    def body(x_ref, idx_ref, w_ref, out_ref, comm_ref, acc_ref, send_sems, recv_sems):
        my = lax.axis_index("i")
        left = lax.rem(my + N_DEV - 1, N_DEV)
        right = lax.rem(my + 1, N_DEV)

        barrier_sem = pltpu.get_barrier_semaphore()
        for nbr in (left, right):
            pl.semaphore_signal(
                barrier_sem, inc=1,
                device_id=(nbr,), device_id_type=pl.DeviceIdType.MESH,
            )
        pl.semaphore_wait(barrier_sem, 2)

        def partial_chunk(c):
            xc = x_ref[pl.ds(c * CHUNK, CHUNK), :]
            ic = idx_ref[pl.ds(c * CHUNK, CHUNK), :]
            acc_ref[...] = jnp.zeros_like(acc_ref)
            for le in range(E_LOCAL):
                e = my * E_LOCAL + le
                xm = jnp.where(ic == e, xc, jnp.zeros_like(xc))
                acc_ref[...] += jnp.dot(
                    xm, w_ref[le], preferred_element_type=jnp.float32
                )

        partial_chunk(my)
        comm_ref[0, :, :] = acc_ref[...].astype(jnp.bfloat16)

        for s in range(N_STEPS):
            send_slot = s % 3
            recv_slot = (s + 1) % 3
            rdma = pltpu.make_async_remote_copy(
                src_ref=comm_ref.at[send_slot],
                dst_ref=comm_ref.at[recv_slot],
                send_sem=send_sems.at[s],
                recv_sem=recv_sems.at[s],
                device_id=(right,),
                device_id_type=pl.DeviceIdType.MESH,
            )
            rdma.start()
            rdma.wait()

            if s < N_DEV - 1:
                cr = lax.rem(my - 1 - s + 2 * N_DEV, N_DEV)
                partial_chunk(cr)
                comm_ref[recv_slot, :, :] = (
                    comm_ref[recv_slot, :, :].astype(jnp.float32) + acc_ref[...]
                ).astype(jnp.bfloat16)
                if s == N_DEV - 2:
                    out_ref[pl.ds(cr * CHUNK, CHUNK), :] = comm_ref[recv_slot, :, :]
            else:
                t = s - (N_DEV - 1)
                cr = lax.rem(my - t + N_DEV, N_DEV)
                out_ref[pl.ds(cr * CHUNK, CHUNK), :] = comm_ref[recv_slot, :, :]

    out = pl.pallas_call(
        body,
        out_shape=jax.ShapeDtypeStruct((N_TOK, H), jnp.bfloat16),
        in_specs=[
            pl.BlockSpec(memory_space=pltpu.VMEM),
            pl.BlockSpec(memory_space=pltpu.VMEM),
            pl.BlockSpec(memory_space=pltpu.VMEM),
        ],
        out_specs=pl.BlockSpec(memory_space=pltpu.VMEM),
        scratch_shapes=[
            pltpu.VMEM((3, CHUNK, H), jnp.bfloat16),
            pltpu.VMEM((CHUNK, H), jnp.float32),
            pltpu.SemaphoreType.DMA((N_STEPS,)),
            pltpu.SemaphoreType.DMA((N_STEPS,)),
        ],
        compiler_params=pltpu.CompilerParams(collective_id=0),
    )(xb, route_idx, wb)
    return out.astype(jnp.float32)


# baseline (device time: 421631 ns/iter reference)
import jax
import jax.numpy as jnp
from jax import lax
from jax.experimental import pallas as pl
from jax.experimental.pallas import tpu as pltpu

N_DEV = 4
N_TOK = 4096
D_MODEL = 1024
H = 2048
HB = H // 2
E_LOCAL = 4
CHUNK = N_TOK // N_DEV
N_STEPS = 2 * (N_DEV - 1)


def kernel(x, router_W, route_idx, expert_W):
    del router_W
    xb = x.astype(jnp.bfloat16)
    wb = expert_W.astype(jnp.bfloat16)

    def body(x_ref, idx_ref, w_ref, out_ref, comm_ref, xm_ref,
             send_sems, recv_sems, copy_sem):
        my = lax.axis_index("i")
        left = lax.rem(my + N_DEV - 1, N_DEV)
        right = lax.rem(my + 1, N_DEV)

        barrier_sem = pltpu.get_barrier_semaphore()
        for nbr in (left, right):
            pl.semaphore_signal(
                barrier_sem, inc=1,
                device_id=(nbr,), device_id_type=pl.DeviceIdType.MESH,
            )
        pl.semaphore_wait(barrier_sem, 2)

        def add_partial(c, slot, seed):
            xc = x_ref[pl.ds(c * CHUNK, CHUNK), :]
            ic = idx_ref[pl.ds(c * CHUNK, CHUNK), :]
            for le in range(E_LOCAL):
                e = my * E_LOCAL + le
                xm_ref[...] = jnp.where(ic == e, xc, jnp.zeros_like(xc))
                for hb in range(H // HB):
                    val = jnp.dot(
                        xm_ref[...],
                        w_ref[le, :, pl.ds(hb * HB, HB)],
                        preferred_element_type=jnp.float32,
                    ).astype(jnp.bfloat16)
                    dst = (slot, slice(None), pl.ds(hb * HB, HB))
                    if seed and le == 0:
                        comm_ref[dst] = val
                    else:
                        comm_ref[dst] += val

        def store_chunk(c, slot):
            cp = pltpu.make_async_copy(
                comm_ref.at[slot],
                out_ref.at[pl.ds(c * CHUNK, CHUNK), :],
                copy_sem,
            )
            cp.start()
            cp.wait()

        add_partial(my, 0, seed=True)

        for s in range(N_STEPS):
            send_slot = s % 3
            recv_slot = (s + 1) % 3
            rdma = pltpu.make_async_remote_copy(
                src_ref=comm_ref.at[send_slot],
                dst_ref=comm_ref.at[recv_slot],
                send_sem=send_sems.at[s],
                recv_sem=recv_sems.at[s],
                device_id=(right,),
                device_id_type=pl.DeviceIdType.MESH,
            )
            rdma.start()
            rdma.wait()

            if s < N_DEV - 1:
                cr = lax.rem(my - 1 - s + 2 * N_DEV, N_DEV)
                add_partial(cr, recv_slot, seed=False)
                if s == N_DEV - 2:
                    store_chunk(cr, recv_slot)
            else:
                t = s - (N_DEV - 1)
                cr = lax.rem(my - t + N_DEV, N_DEV)
                store_chunk(cr, recv_slot)

    out = pl.pallas_call(
        body,
        out_shape=jax.ShapeDtypeStruct((N_TOK, H), jnp.bfloat16),
        in_specs=[
            pl.BlockSpec(memory_space=pltpu.VMEM),
            pl.BlockSpec(memory_space=pltpu.VMEM),
            pl.BlockSpec(memory_space=pltpu.VMEM),
        ],
        out_specs=pl.BlockSpec(memory_space=pl.ANY),
        scratch_shapes=[
            pltpu.VMEM((3, CHUNK, H), jnp.bfloat16),
            pltpu.VMEM((CHUNK, D_MODEL), jnp.bfloat16),
            pltpu.SemaphoreType.DMA((N_STEPS,)),
            pltpu.SemaphoreType.DMA((N_STEPS,)),
            pltpu.SemaphoreType.DMA,
        ],
        compiler_params=pltpu.CompilerParams(
            collective_id=0, vmem_limit_bytes=100 * 1024 * 1024
        ),
    )(xb, route_idx, wb)
    return out.astype(jnp.float32)


# device time: 232496 ns/iter; 1.8135x vs baseline; 1.8135x over previous
import jax
import jax.numpy as jnp
from jax import lax
from jax.experimental import pallas as pl
from jax.experimental.pallas import tpu as pltpu

N_DEV = 4
N_TOK = 4096
D_MODEL = 1024
H = 2048
HB = H // 2
E_LOCAL = 4
CHUNK = N_TOK // N_DEV
N_STEPS = 2 * (N_DEV - 1)


def kernel(x, router_W, route_idx, expert_W):
    del router_W
    xb = x.astype(jnp.bfloat16)
    wb = expert_W.astype(jnp.bfloat16)

    def body(x_ref, idx_ref, w_ref, out_ref, comm_cw, comm_ccw, xm_ref,
             acc_cw, acc_ccw, send_cw, recv_cw, send_ccw, recv_ccw, copy_sems):
        my = lax.axis_index("i")
        left = lax.rem(my + N_DEV - 1, N_DEV)
        right = lax.rem(my + 1, N_DEV)

        barrier_sem = pltpu.get_barrier_semaphore()
        for nbr in (left, right):
            pl.semaphore_signal(
                barrier_sem, inc=1,
                device_id=(nbr,), device_id_type=pl.DeviceIdType.MESH,
            )
        pl.semaphore_wait(barrier_sem, 2)

        def accum(dst, half, le, first):
            val = jnp.dot(
                xm_ref[...],
                w_ref[le, :, pl.ds(half * HB, HB)],
                preferred_element_type=jnp.float32,
            ).astype(jnp.bfloat16)
            if first:
                dst[...] = val
            else:
                dst[...] += val

        def partial_one(c, dst, half):
            xc = x_ref[pl.ds(c * CHUNK, CHUNK), :]
            ic = idx_ref[pl.ds(c * CHUNK, CHUNK), :]
            for le in range(E_LOCAL):
                xm_ref[...] = jnp.where(ic == my * E_LOCAL + le, xc,
                                        jnp.zeros_like(xc))
                accum(dst, half, le, le == 0)

        def partial_pair(c_cw, c_ccw, shared, dst_cw, dst_ccw):
            if shared:
                xc = x_ref[pl.ds(c_cw * CHUNK, CHUNK), :]
                ic = idx_ref[pl.ds(c_cw * CHUNK, CHUNK), :]
                for le in range(E_LOCAL):
                    xm_ref[...] = jnp.where(ic == my * E_LOCAL + le, xc,
                                            jnp.zeros_like(xc))
                    accum(dst_cw, 0, le, le == 0)
                    accum(dst_ccw, 1, le, le == 0)
            else:
                partial_one(c_cw, dst_cw, 0)
                partial_one(c_ccw, dst_ccw, 1)

        def store_half(c, src, half, sem_idx):
            cp = pltpu.make_async_copy(
                src,
                out_ref.at[pl.ds(c * CHUNK, CHUNK), pl.ds(half * HB, HB)],
                copy_sems.at[sem_idx],
            )
            cp.start()
            return cp

        partial_pair(my, my, True, comm_cw.at[0], comm_ccw.at[0])

        for s in range(N_STEPS):
            send_slot = s % 3
            recv_slot = (s + 1) % 3
            rdma_cw = pltpu.make_async_remote_copy(
                src_ref=comm_cw.at[send_slot],
                dst_ref=comm_cw.at[recv_slot],
                send_sem=send_cw.at[s],
                recv_sem=recv_cw.at[s],
                device_id=(right,),
                device_id_type=pl.DeviceIdType.MESH,
            )
            rdma_ccw = pltpu.make_async_remote_copy(
                src_ref=comm_ccw.at[send_slot],
                dst_ref=comm_ccw.at[recv_slot],
                send_sem=send_ccw.at[s],
                recv_sem=recv_ccw.at[s],
                device_id=(left,),
                device_id_type=pl.DeviceIdType.MESH,
            )
            rdma_cw.start()
            rdma_ccw.start()

            if s < N_DEV - 1:
                cr_cw = lax.rem(my - 1 - s + 2 * N_DEV, N_DEV)
                cr_ccw = lax.rem(my + 1 + s, N_DEV)
                partial_pair(cr_cw, cr_ccw, s % 2 == 1, acc_cw, acc_ccw)
                rdma_cw.wait()
                rdma_ccw.wait()
                comm_cw[recv_slot, :, :] += acc_cw[...]
                comm_ccw[recv_slot, :, :] += acc_ccw[...]
                if s == N_DEV - 2:
                    cp0 = store_half(cr_cw, comm_cw.at[recv_slot], 0, 0)
                    cp1 = store_half(cr_ccw, comm_ccw.at[recv_slot], 1, 1)
                    cp0.wait()
                    cp1.wait()
            else:
                t = s - (N_DEV - 1)
                cr_cw = lax.rem(my - t + N_DEV, N_DEV)
                cr_ccw = lax.rem(my + t, N_DEV)
                rdma_cw.wait()
                rdma_ccw.wait()
                cp0 = store_half(cr_cw, comm_cw.at[recv_slot], 0, 0)
                cp1 = store_half(cr_ccw, comm_ccw.at[recv_slot], 1, 1)
                cp0.wait()
                cp1.wait()

    out = pl.pallas_call(
        body,
        out_shape=jax.ShapeDtypeStruct((N_TOK, H), jnp.bfloat16),
        in_specs=[
            pl.BlockSpec(memory_space=pltpu.VMEM),
            pl.BlockSpec(memory_space=pltpu.VMEM),
            pl.BlockSpec(memory_space=pltpu.VMEM),
        ],
        out_specs=pl.BlockSpec(memory_space=pl.ANY),
        scratch_shapes=[
            pltpu.VMEM((3, CHUNK, HB), jnp.bfloat16),
            pltpu.VMEM((3, CHUNK, HB), jnp.bfloat16),
            pltpu.VMEM((CHUNK, D_MODEL), jnp.bfloat16),
            pltpu.VMEM((CHUNK, HB), jnp.bfloat16),
            pltpu.VMEM((CHUNK, HB), jnp.bfloat16),
            pltpu.SemaphoreType.DMA((N_STEPS,)),
            pltpu.SemaphoreType.DMA((N_STEPS,)),
            pltpu.SemaphoreType.DMA((N_STEPS,)),
            pltpu.SemaphoreType.DMA((N_STEPS,)),
            pltpu.SemaphoreType.DMA((2,)),
        ],
        compiler_params=pltpu.CompilerParams(
            collective_id=0, vmem_limit_bytes=100 * 1024 * 1024
        ),
    )(xb, route_idx, wb)
    return out.astype(jnp.float32)
